# baseline (speedup 1.0000x reference)
import jax
import jax.numpy as jnp
from jax.experimental import pallas as pl
from jax.experimental.pallas import tpu as pltpu


def _mlp_kernel(x_ref, w1p_ref, b1p_ref, w2p_ref, b2p_ref, o_ref):
    tb = x_ref.shape[0]
    tb2 = tb // 2
    tb4 = tb // 4
    kin = x_ref.shape[1]
    pad = 128 - kin

    xa = x_ref[0:tb2, :]
    xb = x_ref[tb2:, :]
    x2 = jnp.concatenate(
        [
            jnp.pad(xa, ((0, 0), (0, pad))),
            jnp.pad(xb, ((0, 0), (0, pad))),
        ],
        axis=1,
    )
    h2 = jnp.dot(x2, w1p_ref[...], preferred_element_type=jnp.float32)
    h2 = jnp.maximum(h2 + b1p_ref[...], 0.0)

    l2 = jnp.concatenate([h2[0:tb4, 0:128], h2[tb4:, 0:128]], axis=1)
    y4 = jnp.dot(l2, w2p_ref[...], preferred_element_type=jnp.float32)
    y4 = y4 + b2p_ref[...]

    o = o_ref.shape[1]
    o_ref[0:tb4, :] = y4[:, 0:o]
    o_ref[tb2 : tb2 + tb4, :] = y4[:, 64 : 64 + o]
    o_ref[tb4:tb2, :] = y4[:, 128 : 128 + o]
    o_ref[tb2 + tb4 :, :] = y4[:, 192 : 192 + o]


def kernel(x, w1, b1, w2, b2, *, block_batch=4096):
    B, K = x.shape
    H = w1.shape[1]
    O = w2.shape[1]

    w1p = (
        jnp.zeros((256, 256), jnp.float32)
        .at[0:K, 0:H]
        .set(w1)
        .at[128 : 128 + K, H : 2 * H]
        .set(w1)
    )
    b1p = jnp.zeros((1, 256), jnp.float32).at[:, 0:H].set(b1).at[:, H : 2 * H].set(b1)
    w2p = jnp.zeros((256, 256), jnp.float32)
    b2p = jnp.zeros((1, 256), jnp.float32)
    for g in range(4):
        w2p = w2p.at[g * H : g * H + H, g * 64 : g * 64 + O].set(w2)
        b2p = b2p.at[:, g * 64 : g * 64 + O].set(b2)

    TB = min(block_batch, B)
    grid = (pl.cdiv(B, TB),)

    cost = pl.CostEstimate(
        flops=2 * B * (K * H + H * O),
        transcendentals=0,
        bytes_accessed=4 * (B * (K + O) + 2 * 256 * 256 + 2 * 256),
    )

    return pl.pallas_call(
        _mlp_kernel,
        out_shape=jax.ShapeDtypeStruct((B, O), jnp.float32),
        grid=grid,
        in_specs=[
            pl.BlockSpec((TB, K), lambda i: (i, 0)),
            pl.BlockSpec((256, 256), lambda i: (0, 0)),
            pl.BlockSpec((1, 256), lambda i: (0, 0)),
            pl.BlockSpec((256, 256), lambda i: (0, 0)),
            pl.BlockSpec((1, 256), lambda i: (0, 0)),
        ],
        out_specs=pl.BlockSpec((TB, O), lambda i: (i, 0)),
        compiler_params=pltpu.CompilerParams(
            dimension_semantics=("parallel",)
        ),
        cost_estimate=cost,
    )(x, w1p, b1p, w2p, b2p)


# speedup vs baseline: 1.0724x; 1.0724x over previous
import jax
import jax.numpy as jnp
from jax.experimental import pallas as pl
from jax.experimental.pallas import tpu as pltpu


def _mlp_kernel(x_ref, w1p_ref, b1p_ref, w2p_ref, b2p_ref, o_ref):
    tb = x_ref.shape[0]
    tb2 = tb // 2
    tb4 = tb // 4
    kin = x_ref.shape[1]
    pad = 128 - kin

    xa = x_ref[0:tb2, :]
    xb = x_ref[tb2:, :]
    x2 = jnp.concatenate(
        [
            jnp.pad(xa, ((0, 0), (0, pad))),
            jnp.pad(xb, ((0, 0), (0, pad))),
        ],
        axis=1,
    )
    h2 = jnp.dot(x2, w1p_ref[...], preferred_element_type=jnp.float32)
    h2 = jnp.maximum(h2 + b1p_ref[...], 0.0)

    l2 = jnp.concatenate([h2[0:tb4, 0:128], h2[tb4:, 0:128]], axis=1)
    y4 = jnp.dot(l2, w2p_ref[...], preferred_element_type=jnp.float32)
    y4 = y4 + b2p_ref[...]

    o = o_ref.shape[1]
    o_ref[0:tb4, :] = y4[:, 0:o]
    o_ref[tb2 : tb2 + tb4, :] = y4[:, 64 : 64 + o]
    o_ref[tb4:tb2, :] = y4[:, 128 : 128 + o]
    o_ref[tb2 + tb4 :, :] = y4[:, 192 : 192 + o]


def kernel(x, w1, b1, w2, b2, *, block_batch=16384):
    B, K = x.shape
    H = w1.shape[1]
    O = w2.shape[1]

    w1p = (
        jnp.zeros((256, 256), jnp.float32)
        .at[0:K, 0:H]
        .set(w1)
        .at[128 : 128 + K, H : 2 * H]
        .set(w1)
    )
    b1p = jnp.zeros((1, 256), jnp.float32).at[:, 0:H].set(b1).at[:, H : 2 * H].set(b1)
    w2p = jnp.zeros((256, 256), jnp.float32)
    b2p = jnp.zeros((1, 256), jnp.float32)
    for g in range(4):
        w2p = w2p.at[g * H : g * H + H, g * 64 : g * 64 + O].set(w2)
        b2p = b2p.at[:, g * 64 : g * 64 + O].set(b2)

    TB = min(block_batch, B)
    grid = (pl.cdiv(B, TB),)

    cost = pl.CostEstimate(
        flops=2 * B * (K * H + H * O),
        transcendentals=0,
        bytes_accessed=4 * (B * (K + O) + 2 * 256 * 256 + 2 * 256),
    )

    return pl.pallas_call(
        _mlp_kernel,
        out_shape=jax.ShapeDtypeStruct((B, O), jnp.float32),
        grid=grid,
        in_specs=[
            pl.BlockSpec((TB, K), lambda i: (i, 0)),
            pl.BlockSpec((256, 256), lambda i: (0, 0)),
            pl.BlockSpec((1, 256), lambda i: (0, 0)),
            pl.BlockSpec((256, 256), lambda i: (0, 0)),
            pl.BlockSpec((1, 256), lambda i: (0, 0)),
        ],
        out_specs=pl.BlockSpec((TB, O), lambda i: (i, 0)),
        compiler_params=pltpu.CompilerParams(
            dimension_semantics=("parallel",)
        ),
        cost_estimate=cost,
    )(x, w1p, b1p, w2p, b2p)
